# baseline (speedup 1.0000x reference)
import functools

import jax
import jax.numpy as jnp
import numpy as np
from jax import lax
from jax.experimental import pallas as pl
from jax.experimental.pallas import tpu as pltpu
from jax.experimental.pallas import tpu_sc as plsc

N = 10000
E = 320000
D = 128

NC = 2
NS = 16
NW = NC * NS
L = 16

NP = 10240
NCHUNK_N = NP // 128

C = 32
CHT = 320
NSTAGE = 8
CHH = CHT // NSTAGE
EPAD = NW * CHT * C
CCAP = CHT * C + C
DW = D + L

_mesh = plsc.VectorSubcoreMesh(
    core_axis_name="c", subcore_axis_name="s", num_cores=NC, num_subcores=NS)
_sc_params = pltpu.CompilerParams(needs_layout_passes=False)


def _gather_body(uemb, uidx, iemb, iidx, xu, xi, idxb, rowsb, sem):
  cid = lax.axis_index("c")
  sid = lax.axis_index("s")
  wid = sid * NC + cid

  def one(emb_hbm, idx_hbm, out_hbm):
    pltpu.sync_copy(idx_hbm, idxb)
    for k in range(3):
      c = wid + k * NW

      @pl.when(c < NCHUNK_N)
      def _():
        pltpu.async_copy(emb_hbm.at[idxb.at[c]], rowsb, sem).wait()
        pltpu.sync_copy(rowsb, out_hbm.at[pl.ds(c * 128, 128)])

  one(uemb, uidx, xu)
  one(iemb, iidx, xi)


_gather_call = pl.kernel(
    _gather_body,
    out_type=[
        jax.ShapeDtypeStruct((NP, D), jnp.float32),
        jax.ShapeDtypeStruct((NP, D), jnp.float32),
    ],
    mesh=_mesh,
    scratch_types=[
        pltpu.VMEM((NCHUNK_N, 128), jnp.int32),
        pltpu.VMEM((128, D), jnp.float32),
        pltpu.SemaphoreType.DMA,
    ],
    compiler_params=_sc_params,
)


def _prep_body(xu, xi, Wsu, Wdu, atsu, atdu, Wsi, Wdi, atsi, atdi,
               hu, asu, adu, hi, asi, adi):
  def conv(x, Ws, Wd, ats, atd, h_o, as_o, ad_o):
    h = jnp.dot(x[...], Ws[...], preferred_element_type=jnp.float32)
    h_o[...] = h
    as_o[...] = jnp.dot(h, ats[...], preferred_element_type=jnp.float32)
    wd = jnp.dot(Wd[...], atd[...], preferred_element_type=jnp.float32)
    ad_o[...] = jnp.dot(x[...], wd, preferred_element_type=jnp.float32)

  conv(xu, Wsu, Wdu, atsu, atdu, hu, asu, adu)
  conv(xi, Wsi, Wdi, atsi, atdi, hi, asi, adi)


def _stage_b(xu, xi, Wsu, Wdu, atsu, atdu, Wsi, Wdi, atsi, atdi):
  BN = 640
  G = NP // BN
  row = lambda i: (i, 0)
  fixw = pl.BlockSpec((D, D), lambda i: (0, 0))
  fixv = pl.BlockSpec((D, 1), lambda i: (0, 0))
  return pl.pallas_call(
      _prep_body,
      grid=(G,),
      in_specs=[
          pl.BlockSpec((BN, D), row), pl.BlockSpec((BN, D), row),
          fixw, fixw, fixv, fixv, fixw, fixw, fixv, fixv,
      ],
      out_specs=[
          pl.BlockSpec((BN, D), row), pl.BlockSpec((BN, 1), row),
          pl.BlockSpec((BN, 1), row),
          pl.BlockSpec((BN, D), row), pl.BlockSpec((BN, 1), row),
          pl.BlockSpec((BN, 1), row),
      ],
      out_shape=[
          jax.ShapeDtypeStruct((NP, D), jnp.float32),
          jax.ShapeDtypeStruct((NP, 1), jnp.float32),
          jax.ShapeDtypeStruct((NP, 1), jnp.float32),
          jax.ShapeDtypeStruct((NP, D), jnp.float32),
          jax.ShapeDtypeStruct((NP, 1), jnp.float32),
          jax.ShapeDtypeStruct((NP, 1), jnp.float32),
      ],
  )(xu, xi, Wsu, Wdu, atsu, atdu, Wsi, Wdi, atsi, atdi)


NH = NP // 2


NSPLIT = 2


def _edge_body(hu, asu, adu, su, du, hi, asi, adi, si, di,
               nu, dnu, ni, dni,
               asrc_t, adst_t, den_t, sball, dball, csrc, cw, cdadj,
               dadj0, dadj1, rows0, rows1, num_sp, gsem0, gsem1, ssem0, ssem1):
  cid = lax.axis_index("c")
  sid = lax.axis_index("s")
  wid = sid * NC + cid
  zeros16 = jnp.zeros((L,), jnp.float32)
  bufs = ((dadj0, rows0, gsem0, ssem0),
          (dadj1, rows1, gsem1, ssem1))
  Q = C // NSPLIT

  def run_conv(h_hbm, asrc_hbm, adst_hbm, src_hbm, dst_hbm, num_out, den_out):
    pltpu.sync_copy(asrc_hbm, asrc_t)
    pltpu.sync_copy(adst_hbm, adst_t)

    def zden(j, carry):
      den_t[pl.ds(j * L, L)] = zeros16
      return carry
    lax.fori_loop(0, NP // L, zden, 0)

    for ph in range(2):
      base = ph * NH

      def zrow(r, carry):
        for k in range(D // L):
          rows0[r, pl.ds(k * L, L)] = zeros16
        return carry
      lax.fori_loop(0, C, zrow, 0)
      for t in range(NH // NS // C):
        pltpu.sync_copy(rows0, num_sp.at[pl.ds(sid * (NH // NS) + t * C, C)])
      plsc.subcore_barrier()

      cnt = jnp.int32(0)
      for half in range(NSTAGE):
        pltpu.sync_copy(src_hbm.at[pl.ds(wid * CHT + half * CHH, CHH)], sball)
        pltpu.sync_copy(dst_hbm.at[pl.ds(wid * CHT + half * CHH, CHH)], dball)

        def scan_c(c2, cnt):
          for j in range(C // L):
            s16 = sball[c2, pl.ds(j * L, L)]
            d16 = dball[c2, pl.ds(j * L, L)]
            a = (plsc.load_gather(asrc_t, [s16])
                 + plsc.load_gather(adst_t, [d16]))
            e = jnp.where(a > 0, a, 0.2 * a)
            w = jnp.exp(e)
            pos = ((wid * CHT + half * CHH + c2) * C + j * L
                   + lax.broadcasted_iota(jnp.int32, (L,), 0))
            real = pos < E
            w = jnp.where(real, w, 0.0)
            if ph == 0:
              plsc.addupdate_scatter(den_t, [d16], w)
            drel = d16 - base
            inr = (drel >= 0) & (drel < NH) & real
            plsc.store_compressed(csrc.at[pl.ds(cnt, L)], s16, mask=inr)
            plsc.store_compressed(cw.at[pl.ds(cnt, L)], w, mask=inr)
            plsc.store_compressed(cdadj.at[pl.ds(cnt, L)], drel, mask=inr)
            cnt = cnt + plsc.all_reduce_population_count(inr)[0]
          return cnt
        cnt = lax.fori_loop(0, CHH, scan_c, cnt)

      for k in range(C // L):
        csrc[pl.ds(cnt + k * L, L)] = jnp.zeros((L,), jnp.int32)
        cw[pl.ds(cnt + k * L, L)] = zeros16
        cdadj[pl.ds(cnt + k * L, L)] = jnp.zeros((L,), jnp.int32)
      nch = (cnt + C - 1) // C

      def rscale(c, rows):
        def rgrp(j, carry2):
          wv = cw[pl.ds(c * C + j * L, L)]
          for t in range(L):
            r = j * L + t
            w_r = wv[t]
            for k in range(D // L):
              rows[r, pl.ds(k * L, L)] = rows[r, pl.ds(k * L, L)] * w_r
          return carry2
        lax.fori_loop(0, C // L, rgrp, 0)

      def gissue(c, rows, gsem):
        for q in range(NSPLIT):
          pltpu.async_copy(h_hbm.at[csrc.at[pl.ds(c * C + q * Q, Q)]],
                           rows.at[pl.ds(q * Q, Q)], gsem)

      def gwait(c, rows, gsem):
        for q in range(NSPLIT):
          pltpu.make_async_copy(h_hbm.at[csrc.at[pl.ds(c * C + q * Q, Q)]],
                                rows.at[pl.ds(q * Q, Q)], gsem).wait()

      def load_dadj(c, dadj):
        for k in range(C // L):
          dadj[pl.ds(k * L, L)] = cdadj[pl.ds(c * C + k * L, L)]

      pl.when(nch > 0)(lambda: gissue(0, rows0, gsem0))

      def pair(g, carry):
        for b in range(2):
          dadj, rows, gsem, ssem = bufs[b]
          odadj, orows, ogsem, ossem = bufs[b ^ 1]
          c = 2 * g + b

          def drain(_=None):
            pltpu.make_async_copy(orows, num_sp.at[odadj], ossem).wait()
          def issue(_=None):
            gissue(c + 1, orows, ogsem)
          if b == 0:
            pl.when((g >= 1) & (c - 1 < nch))(drain)
          else:
            pl.when(c - 1 < nch)(drain)
          pl.when(c + 1 < nch)(issue)

          @pl.when(c < nch)
          def _():
            load_dadj(c, dadj)
            gwait(c, rows, gsem)
            rscale(c, rows)
            pltpu.async_copy(rows, num_sp.at[dadj], ssem, add=True)
        return carry
      lax.fori_loop(0, CHT // 2, pair, 0)
      pl.when(nch == CHT)(
          lambda: pltpu.make_async_copy(rows1, num_sp.at[dadj1], ssem1).wait())

      plsc.subcore_barrier()
      pltpu.sync_copy(
          num_sp.at[pl.ds(sid * (NH // NS), NH // NS)],
          num_out.at[cid, pl.ds(base + sid * (NH // NS), NH // NS)])
      plsc.subcore_barrier()
    pltpu.sync_copy(den_t, den_out.at[wid])

  run_conv(hu, asu, adu, su, du, nu, dnu)
  plsc.subcore_barrier()
  run_conv(hi, asi, adi, si, di, ni, dni)


_edge_call = pl.kernel(
    _edge_body,
    out_type=[
        jax.ShapeDtypeStruct((NC, NP, D), jnp.float32),
        jax.ShapeDtypeStruct((NW, NP), jnp.float32),
        jax.ShapeDtypeStruct((NC, NP, D), jnp.float32),
        jax.ShapeDtypeStruct((NW, NP), jnp.float32),
    ],
    mesh=_mesh,
    scratch_types=[
        pltpu.VMEM((NP,), jnp.float32),
        pltpu.VMEM((NP,), jnp.float32),
        pltpu.VMEM((NP,), jnp.float32),
        pltpu.VMEM((CHH, C), jnp.int32),
        pltpu.VMEM((CHH, C), jnp.int32),
        pltpu.VMEM((CCAP,), jnp.int32),
        pltpu.VMEM((CCAP,), jnp.float32),
        pltpu.VMEM((CCAP,), jnp.int32),
        pltpu.VMEM((C,), jnp.int32),
        pltpu.VMEM((C,), jnp.int32),
        pltpu.VMEM((C, D), jnp.float32),
        pltpu.VMEM((C, D), jnp.float32),
        pltpu.VMEM_SHARED((NH, D), jnp.float32),
        pltpu.SemaphoreType.DMA,
        pltpu.SemaphoreType.DMA,
        pltpu.SemaphoreType.DMA,
        pltpu.SemaphoreType.DMA,
    ],
    compiler_params=_sc_params,
)


def _final_body(nu, dnu, hu, asu, adu, ni, dni, hi, asi, adi,
                Wlu, blu, bu, Wli, bli, bi, W1u, W1i, b1, W2, b2, out):
  ones = jnp.ones((NW, 1), jnp.float32)

  def conv(n, dn, h, a_s, a_d, Wl, bl, b):
    a = a_s[...] + a_d[...]
    wself = jnp.exp(jnp.where(a > 0, a, 0.2 * a))
    den = lax.dot_general(dn[...], ones, (((0,), (0,)), ((), ())),
                          preferred_element_type=jnp.float32)
    num = n[0] + n[1] + wself * h[...]
    g = num / (den + wself + 1e-16) + b[...]
    return jnp.maximum(
        jnp.dot(g, Wl[...], preferred_element_type=jnp.float32) + bl[...], 0.0)

  u2 = conv(nu, dnu, hu, asu, adu, Wlu, blu, bu)
  i2 = conv(ni, dni, hi, asi, adi, Wli, bli, bi)
  y = (jnp.dot(u2, W1u[...], preferred_element_type=jnp.float32)
       + jnp.dot(i2, W1i[...], preferred_element_type=jnp.float32) + b1[...])
  z = jnp.dot(y, W2[...], preferred_element_type=jnp.float32) + b2[...]
  out[...] = 1.0 / (1.0 + jnp.exp(-z))


def _stage_d(nu, dnu, hu, asu, adu, ni, dni, hi, asi, adi,
             Wlu, blu, bu, Wli, bli, bi, W1u, W1i, b1, W2, b2):
  BN = 640
  G = NP // BN
  n_spec = pl.BlockSpec((NC, BN, D), lambda i: (0, i, 0))
  dn_spec = pl.BlockSpec((NW, BN), lambda i: (0, i))
  row = lambda i: (i, 0)
  fixw = pl.BlockSpec((D, D), lambda i: (0, 0))
  fixr = pl.BlockSpec((1, D), lambda i: (0, 0))
  return pl.pallas_call(
      _final_body,
      grid=(G,),
      in_specs=[
          n_spec, dn_spec, pl.BlockSpec((BN, D), row),
          pl.BlockSpec((BN, 1), row), pl.BlockSpec((BN, 1), row),
          n_spec, dn_spec, pl.BlockSpec((BN, D), row),
          pl.BlockSpec((BN, 1), row), pl.BlockSpec((BN, 1), row),
          fixw, fixr, fixr, fixw, fixr, fixr, fixw, fixw, fixr,
          pl.BlockSpec((D, 1), lambda i: (0, 0)),
          pl.BlockSpec((1, 1), lambda i: (0, 0)),
      ],
      out_specs=[pl.BlockSpec((BN, 1), row)],
      out_shape=[jax.ShapeDtypeStruct((NP, 1), jnp.float32)],
  )(nu, dnu, hu, asu, adu, ni, dni, hi, asi, adi,
    Wlu, blu, bu, Wli, bli, bi, W1u, W1i, b1, W2, b2)


def kernel(user_idx, item_idx, edge_index_ui, edge_index_iu, user_emb,
           item_emb, W_src_u, W_dst_u, att_src_u, att_dst_u, bias_u, W_lin_u,
           b_lin_u, W_src_i, W_dst_i, att_src_i, att_dst_i, bias_i, W_lin_i,
           b_lin_i, W1, b1, W2, b2):
  i32 = jnp.int32
  uidx = jnp.pad(user_idx.astype(i32), (0, NP - N)).reshape(NCHUNK_N, 128)
  iidx = jnp.pad(item_idx.astype(i32), (0, NP - N)).reshape(NCHUNK_N, 128)

  def edges2d(ei):
    p = jnp.pad(ei.astype(i32), ((0, 0), (0, EPAD - E)))
    return p[0].reshape(NW * CHT, C), p[1].reshape(NW * CHT, C)

  su, du = edges2d(edge_index_ui)
  si, di = edges2d(edge_index_iu)

  xu, xi = _gather_call(user_emb, uidx, item_emb, iidx)

  hu, asu, adu, hi, asi, adi = _stage_b(
      xu, xi, W_src_u, W_dst_u, att_src_u.reshape(D, 1),
      att_dst_u.reshape(D, 1), W_src_i, W_dst_i, att_src_i.reshape(D, 1),
      att_dst_i.reshape(D, 1))

  nu, dnu, ni, dni = _edge_call(hu, asu.reshape(NP), adu.reshape(NP), su, du,
                                hi, asi.reshape(NP), adi.reshape(NP), si, di)

  (out,) = _stage_d(nu, dnu, hu, asu, adu, ni, dni, hi, asi, adi,
                    W_lin_u, b_lin_u.reshape(1, D), bias_u.reshape(1, D),
                    W_lin_i, b_lin_i.reshape(1, D), bias_i.reshape(1, D),
                    W1[:D], W1[D:], b1.reshape(1, D), W2, b2.reshape(1, 1))
  return out[:N]


# speedup vs baseline: 1.0604x; 1.0604x over previous
import functools

import jax
import jax.numpy as jnp
import numpy as np
from jax import lax
from jax.experimental import pallas as pl
from jax.experimental.pallas import tpu as pltpu
from jax.experimental.pallas import tpu_sc as plsc

N = 10000
E = 320000
D = 128

NC = 2
NS = 16
NW = NC * NS
L = 16

NP = 10240
NCHUNK_N = NP // 128

C = 32
CHT = 320
NSTAGE = 8
CHH = CHT // NSTAGE
EPAD = NW * CHT * C
CTOP = CHT * C + 2 * C
CCAP = CTOP + C
DW = D + L

_mesh = plsc.VectorSubcoreMesh(
    core_axis_name="c", subcore_axis_name="s", num_cores=NC, num_subcores=NS)
_sc_params = pltpu.CompilerParams(needs_layout_passes=False)


def _gather_body(uemb, uidx, iemb, iidx, xu, xi, idxb, rowsb, sem):
  cid = lax.axis_index("c")
  sid = lax.axis_index("s")
  wid = sid * NC + cid

  def one(emb_hbm, idx_hbm, out_hbm):
    pltpu.sync_copy(idx_hbm, idxb)
    for k in range(3):
      c = wid + k * NW

      @pl.when(c < NCHUNK_N)
      def _():
        pltpu.async_copy(emb_hbm.at[idxb.at[c]], rowsb, sem).wait()
        pltpu.sync_copy(rowsb, out_hbm.at[pl.ds(c * 128, 128)])

  one(uemb, uidx, xu)
  one(iemb, iidx, xi)


_gather_call = pl.kernel(
    _gather_body,
    out_type=[
        jax.ShapeDtypeStruct((NP, D), jnp.float32),
        jax.ShapeDtypeStruct((NP, D), jnp.float32),
    ],
    mesh=_mesh,
    scratch_types=[
        pltpu.VMEM((NCHUNK_N, 128), jnp.int32),
        pltpu.VMEM((128, D), jnp.float32),
        pltpu.SemaphoreType.DMA,
    ],
    compiler_params=_sc_params,
)


def _prep_body(xu, xi, Wsu, Wdu, atsu, atdu, Wsi, Wdi, atsi, atdi,
               hu, asu, adu, hi, asi, adi):
  def conv(x, Ws, Wd, ats, atd, h_o, as_o, ad_o):
    h = jnp.dot(x[...], Ws[...], preferred_element_type=jnp.float32)
    h_o[...] = h
    as_o[...] = jnp.dot(h, ats[...], preferred_element_type=jnp.float32)
    wd = jnp.dot(Wd[...], atd[...], preferred_element_type=jnp.float32)
    ad_o[...] = jnp.dot(x[...], wd, preferred_element_type=jnp.float32)

  conv(xu, Wsu, Wdu, atsu, atdu, hu, asu, adu)
  conv(xi, Wsi, Wdi, atsi, atdi, hi, asi, adi)


def _stage_b(xu, xi, Wsu, Wdu, atsu, atdu, Wsi, Wdi, atsi, atdi):
  BN = 640
  G = NP // BN
  row = lambda i: (i, 0)
  fixw = pl.BlockSpec((D, D), lambda i: (0, 0))
  fixv = pl.BlockSpec((D, 1), lambda i: (0, 0))
  return pl.pallas_call(
      _prep_body,
      grid=(G,),
      in_specs=[
          pl.BlockSpec((BN, D), row), pl.BlockSpec((BN, D), row),
          fixw, fixw, fixv, fixv, fixw, fixw, fixv, fixv,
      ],
      out_specs=[
          pl.BlockSpec((BN, D), row), pl.BlockSpec((BN, 1), row),
          pl.BlockSpec((BN, 1), row),
          pl.BlockSpec((BN, D), row), pl.BlockSpec((BN, 1), row),
          pl.BlockSpec((BN, 1), row),
      ],
      out_shape=[
          jax.ShapeDtypeStruct((NP, D), jnp.float32),
          jax.ShapeDtypeStruct((NP, 1), jnp.float32),
          jax.ShapeDtypeStruct((NP, 1), jnp.float32),
          jax.ShapeDtypeStruct((NP, D), jnp.float32),
          jax.ShapeDtypeStruct((NP, 1), jnp.float32),
          jax.ShapeDtypeStruct((NP, 1), jnp.float32),
      ],
  )(xu, xi, Wsu, Wdu, atsu, atdu, Wsi, Wdi, atsi, atdi)


NH = NP // 2


NSPLIT = 2


def _edge_body(hu, asu, adu, su, du, hi, asi, adi, si, di,
               nu, dnu, ni, dni,
               asrc_t, adst_t, den_t, sball, dball, csrc, cw, cdadj,
               dadj0, dadj1, rows0, rows1, num_sp, gsem0, gsem1, ssem0, ssem1):
  cid = lax.axis_index("c")
  sid = lax.axis_index("s")
  wid = sid * NC + cid
  zeros16 = jnp.zeros((L,), jnp.float32)
  bufs = ((dadj0, rows0, gsem0, ssem0),
          (dadj1, rows1, gsem1, ssem1))
  Q = C // NSPLIT

  def run_conv(h_hbm, asrc_hbm, adst_hbm, src_hbm, dst_hbm, num_out, den_out):
    pltpu.sync_copy(asrc_hbm, asrc_t)
    pltpu.sync_copy(adst_hbm, adst_t)

    def zden(j, carry):
      den_t[pl.ds(j * L, L)] = zeros16
      return carry
    lax.fori_loop(0, NP // L, zden, 0)

    cnt0 = jnp.int32(0)
    cnt1 = jnp.int32(0)
    for half in range(NSTAGE):
      pltpu.sync_copy(src_hbm.at[pl.ds(wid * CHT + half * CHH, CHH)], sball)
      pltpu.sync_copy(dst_hbm.at[pl.ds(wid * CHT + half * CHH, CHH)], dball)

      def scan_c(c2, cnts):
        cnt0, cnt1 = cnts
        for j in range(C // L):
          s16 = sball[c2, pl.ds(j * L, L)]
          d16 = dball[c2, pl.ds(j * L, L)]
          a = (plsc.load_gather(asrc_t, [s16])
               + plsc.load_gather(adst_t, [d16]))
          e = jnp.where(a > 0, a, 0.2 * a)
          w = jnp.exp(e)
          pos = ((wid * CHT + half * CHH + c2) * C + j * L
                 + lax.broadcasted_iota(jnp.int32, (L,), 0))
          real = pos < E
          w = jnp.where(real, w, 0.0)
          plsc.addupdate_scatter(den_t, [d16], w)
          inr0 = (d16 < NH) & real
          plsc.store_compressed(csrc.at[pl.ds(cnt0, L)], s16, mask=inr0)
          plsc.store_compressed(cw.at[pl.ds(cnt0, L)], w, mask=inr0)
          plsc.store_compressed(cdadj.at[pl.ds(cnt0, L)], d16, mask=inr0)
          cnt0 = cnt0 + plsc.all_reduce_population_count(inr0)[0]
          inr1 = (d16 >= NH) & real
          pc1 = plsc.all_reduce_population_count(inr1)[0]
          p1 = CTOP - cnt1 - pc1
          plsc.store_compressed(csrc.at[pl.ds(p1, L)], s16, mask=inr1)
          plsc.store_compressed(cw.at[pl.ds(p1, L)], w, mask=inr1)
          plsc.store_compressed(cdadj.at[pl.ds(p1, L)], d16 - NH, mask=inr1)
          cnt1 = cnt1 + pc1
        return (cnt0, cnt1)
      cnt0, cnt1 = lax.fori_loop(0, CHH, scan_c, (cnt0, cnt1))

    for k in range(C // L):
      csrc[pl.ds(cnt0 + k * L, L)] = jnp.zeros((L,), jnp.int32)
      cw[pl.ds(cnt0 + k * L, L)] = zeros16
      cdadj[pl.ds(cnt0 + k * L, L)] = jnp.zeros((L,), jnp.int32)
    s1 = CTOP - cnt1
    astart = (s1 // C) * C
    for k in range(C // L):
      gpos = astart + k * L
      m = gpos + lax.broadcasted_iota(jnp.int32, (L,), 0) < s1
      csrc[pl.ds(gpos, L)] = jnp.where(m, 0, csrc[pl.ds(gpos, L)])
      cw[pl.ds(gpos, L)] = jnp.where(m, 0.0, cw[pl.ds(gpos, L)])
      cdadj[pl.ds(gpos, L)] = jnp.where(m, 0, cdadj[pl.ds(gpos, L)])

    for ph in range(2):
      if ph == 0:
        coff = jnp.int32(0)
        nch = (cnt0 + C - 1) // C
      else:
        coff = astart
        nch = (CTOP - astart) // C

      def zrow(r, carry):
        for k in range(D // L):
          rows0[r, pl.ds(k * L, L)] = zeros16
        return carry
      lax.fori_loop(0, C, zrow, 0)
      for t in range(NH // NS // C):
        pltpu.sync_copy(rows0, num_sp.at[pl.ds(sid * (NH // NS) + t * C, C)])
      plsc.subcore_barrier()

      def rscale(c, rows):
        def rgrp(j, carry2):
          wv = cw[pl.ds(coff + c * C + j * L, L)]
          for t in range(L):
            r = j * L + t
            w_r = wv[t]
            for k in range(D // L):
              rows[r, pl.ds(k * L, L)] = rows[r, pl.ds(k * L, L)] * w_r
          return carry2
        lax.fori_loop(0, C // L, rgrp, 0)

      def gissue(c, rows, gsem):
        for q in range(NSPLIT):
          pltpu.async_copy(h_hbm.at[csrc.at[pl.ds(coff + c * C + q * Q, Q)]],
                           rows.at[pl.ds(q * Q, Q)], gsem)

      def gwait(c, rows, gsem):
        for q in range(NSPLIT):
          pltpu.make_async_copy(h_hbm.at[csrc.at[pl.ds(coff + c * C + q * Q, Q)]],
                                rows.at[pl.ds(q * Q, Q)], gsem).wait()

      def load_dadj(c, dadj):
        for k in range(C // L):
          dadj[pl.ds(k * L, L)] = cdadj[pl.ds(coff + c * C + k * L, L)]

      pl.when(nch > 0)(lambda: gissue(0, rows0, gsem0))

      def pair(g, carry):
        for b in range(2):
          dadj, rows, gsem, ssem = bufs[b]
          odadj, orows, ogsem, ossem = bufs[b ^ 1]
          c = 2 * g + b

          def drain(_=None):
            pltpu.make_async_copy(orows, num_sp.at[odadj], ossem).wait()
          def issue(_=None):
            gissue(c + 1, orows, ogsem)
          if b == 0:
            pl.when((g >= 1) & (c - 1 < nch))(drain)
          else:
            pl.when(c - 1 < nch)(drain)
          pl.when(c + 1 < nch)(issue)

          @pl.when(c < nch)
          def _():
            load_dadj(c, dadj)
            gwait(c, rows, gsem)
            rscale(c, rows)
            pltpu.async_copy(rows, num_sp.at[dadj], ssem, add=True)
        return carry
      lax.fori_loop(0, CHT // 2, pair, 0)
      pl.when(nch == CHT)(
          lambda: pltpu.make_async_copy(rows1, num_sp.at[dadj1], ssem1).wait())

      plsc.subcore_barrier()
      pltpu.sync_copy(
          num_sp.at[pl.ds(sid * (NH // NS), NH // NS)],
          num_out.at[cid, pl.ds(ph * NH + sid * (NH // NS), NH // NS)])
      plsc.subcore_barrier()
    pltpu.sync_copy(den_t, den_out.at[wid])

  run_conv(hu, asu, adu, su, du, nu, dnu)
  plsc.subcore_barrier()
  run_conv(hi, asi, adi, si, di, ni, dni)


_edge_call = pl.kernel(
    _edge_body,
    out_type=[
        jax.ShapeDtypeStruct((NC, NP, D), jnp.float32),
        jax.ShapeDtypeStruct((NW, NP), jnp.float32),
        jax.ShapeDtypeStruct((NC, NP, D), jnp.float32),
        jax.ShapeDtypeStruct((NW, NP), jnp.float32),
    ],
    mesh=_mesh,
    scratch_types=[
        pltpu.VMEM((NP,), jnp.float32),
        pltpu.VMEM((NP,), jnp.float32),
        pltpu.VMEM((NP,), jnp.float32),
        pltpu.VMEM((CHH, C), jnp.int32),
        pltpu.VMEM((CHH, C), jnp.int32),
        pltpu.VMEM((CCAP,), jnp.int32),
        pltpu.VMEM((CCAP,), jnp.float32),
        pltpu.VMEM((CCAP,), jnp.int32),
        pltpu.VMEM((C,), jnp.int32),
        pltpu.VMEM((C,), jnp.int32),
        pltpu.VMEM((C, D), jnp.float32),
        pltpu.VMEM((C, D), jnp.float32),
        pltpu.VMEM_SHARED((NH, D), jnp.float32),
        pltpu.SemaphoreType.DMA,
        pltpu.SemaphoreType.DMA,
        pltpu.SemaphoreType.DMA,
        pltpu.SemaphoreType.DMA,
    ],
    compiler_params=_sc_params,
)


def _final_body(nu, dnu, hu, asu, adu, ni, dni, hi, asi, adi,
                Wlu, blu, bu, Wli, bli, bi, W1u, W1i, b1, W2, b2, out):
  ones = jnp.ones((NW, 1), jnp.float32)

  def conv(n, dn, h, a_s, a_d, Wl, bl, b):
    a = a_s[...] + a_d[...]
    wself = jnp.exp(jnp.where(a > 0, a, 0.2 * a))
    den = lax.dot_general(dn[...], ones, (((0,), (0,)), ((), ())),
                          preferred_element_type=jnp.float32)
    num = n[0] + n[1] + wself * h[...]
    g = num / (den + wself + 1e-16) + b[...]
    return jnp.maximum(
        jnp.dot(g, Wl[...], preferred_element_type=jnp.float32) + bl[...], 0.0)

  u2 = conv(nu, dnu, hu, asu, adu, Wlu, blu, bu)
  i2 = conv(ni, dni, hi, asi, adi, Wli, bli, bi)
  y = (jnp.dot(u2, W1u[...], preferred_element_type=jnp.float32)
       + jnp.dot(i2, W1i[...], preferred_element_type=jnp.float32) + b1[...])
  z = jnp.dot(y, W2[...], preferred_element_type=jnp.float32) + b2[...]
  out[...] = 1.0 / (1.0 + jnp.exp(-z))


def _stage_d(nu, dnu, hu, asu, adu, ni, dni, hi, asi, adi,
             Wlu, blu, bu, Wli, bli, bi, W1u, W1i, b1, W2, b2):
  BN = 640
  G = NP // BN
  n_spec = pl.BlockSpec((NC, BN, D), lambda i: (0, i, 0))
  dn_spec = pl.BlockSpec((NW, BN), lambda i: (0, i))
  row = lambda i: (i, 0)
  fixw = pl.BlockSpec((D, D), lambda i: (0, 0))
  fixr = pl.BlockSpec((1, D), lambda i: (0, 0))
  return pl.pallas_call(
      _final_body,
      grid=(G,),
      in_specs=[
          n_spec, dn_spec, pl.BlockSpec((BN, D), row),
          pl.BlockSpec((BN, 1), row), pl.BlockSpec((BN, 1), row),
          n_spec, dn_spec, pl.BlockSpec((BN, D), row),
          pl.BlockSpec((BN, 1), row), pl.BlockSpec((BN, 1), row),
          fixw, fixr, fixr, fixw, fixr, fixr, fixw, fixw, fixr,
          pl.BlockSpec((D, 1), lambda i: (0, 0)),
          pl.BlockSpec((1, 1), lambda i: (0, 0)),
      ],
      out_specs=[pl.BlockSpec((BN, 1), row)],
      out_shape=[jax.ShapeDtypeStruct((NP, 1), jnp.float32)],
  )(nu, dnu, hu, asu, adu, ni, dni, hi, asi, adi,
    Wlu, blu, bu, Wli, bli, bi, W1u, W1i, b1, W2, b2)


def kernel(user_idx, item_idx, edge_index_ui, edge_index_iu, user_emb,
           item_emb, W_src_u, W_dst_u, att_src_u, att_dst_u, bias_u, W_lin_u,
           b_lin_u, W_src_i, W_dst_i, att_src_i, att_dst_i, bias_i, W_lin_i,
           b_lin_i, W1, b1, W2, b2):
  i32 = jnp.int32
  uidx = jnp.pad(user_idx.astype(i32), (0, NP - N)).reshape(NCHUNK_N, 128)
  iidx = jnp.pad(item_idx.astype(i32), (0, NP - N)).reshape(NCHUNK_N, 128)

  def edges2d(ei):
    p = jnp.pad(ei.astype(i32), ((0, 0), (0, EPAD - E)))
    return p[0].reshape(NW * CHT, C), p[1].reshape(NW * CHT, C)

  su, du = edges2d(edge_index_ui)
  si, di = edges2d(edge_index_iu)

  xu, xi = _gather_call(user_emb, uidx, item_emb, iidx)

  hu, asu, adu, hi, asi, adi = _stage_b(
      xu, xi, W_src_u, W_dst_u, att_src_u.reshape(D, 1),
      att_dst_u.reshape(D, 1), W_src_i, W_dst_i, att_src_i.reshape(D, 1),
      att_dst_i.reshape(D, 1))

  nu, dnu, ni, dni = _edge_call(hu, asu.reshape(NP), adu.reshape(NP), su, du,
                                hi, asi.reshape(NP), adi.reshape(NP), si, di)

  (out,) = _stage_d(nu, dnu, hu, asu, adu, ni, dni, hi, asi, adi,
                    W_lin_u, b_lin_u.reshape(1, D), bias_u.reshape(1, D),
                    W_lin_i, b_lin_i.reshape(1, D), bias_i.reshape(1, D),
                    W1[:D], W1[D:], b1.reshape(1, D), W2, b2.reshape(1, 1))
  return out[:N]
